# Initial kernel scaffold; baseline (speedup 1.0000x reference)
import jax
import jax.numpy as jnp
from jax.experimental import pallas as pl


def kernel(x, edge_index, W):
    raise NotImplementedError("write your pallas kernel here")


# speedup vs baseline: 7.8148x; 7.8148x over previous
import functools

import jax
import jax.numpy as jnp
from jax import lax
from jax.experimental import pallas as pl
from jax.experimental.pallas import tpu as pltpu
from jax.experimental.pallas import tpu_sc as plsc

N = 10000
E = 320000
D = 128

NUM_CORES = 2
NUM_SUBCORES = 16
NW = NUM_CORES * NUM_SUBCORES
EPW = E // NW
K = 125
C = EPW // K
ROWS_PER_TILE = 624
TAIL_ROWS = N - NUM_SUBCORES * ROWS_PER_TILE


def _sc_aggregate(x, rows3, cols3, zeros):
    mesh = plsc.VectorSubcoreMesh(core_axis_name="c", subcore_axis_name="s")

    @functools.partial(
        pl.kernel,
        out_type=jax.ShapeDtypeStruct((NUM_CORES, N, D), jnp.float32),
        mesh=mesh,
        scratch_types=[
            pltpu.VMEM((C, K), jnp.int32),
            pltpu.VMEM((C, K), jnp.int32),
            pltpu.VMEM((K, D), jnp.float32),
            pltpu.VMEM_SHARED((N, D), jnp.float32),
            pltpu.SemaphoreType.DMA,
        ],
    )
    def k(x_hbm, rows_hbm, cols_hbm, zeros_hbm, part_hbm,
          colv, rowv, gbuf, acc, sem):
        cid = lax.axis_index("c")
        sid = lax.axis_index("s")
        wid = cid * NUM_SUBCORES + sid

        r0 = sid * ROWS_PER_TILE
        pltpu.sync_copy(zeros_hbm.at[pl.ds(r0, ROWS_PER_TILE)],
                        acc.at[pl.ds(r0, ROWS_PER_TILE)])

        @pl.when(sid == NUM_SUBCORES - 1)
        def _():
            t0 = NUM_SUBCORES * ROWS_PER_TILE
            pltpu.sync_copy(zeros_hbm.at[pl.ds(t0, TAIL_ROWS)],
                            acc.at[pl.ds(t0, TAIL_ROWS)])

        plsc.subcore_barrier()

        pltpu.sync_copy(cols_hbm.at[wid], colv)
        pltpu.sync_copy(rows_hbm.at[wid], rowv)

        def body(j, carry):
            pltpu.async_copy(x_hbm.at[colv.at[j]], gbuf, sem).wait()
            pltpu.sync_copy(gbuf, acc.at[rowv.at[j]], add=True)
            return carry

        lax.fori_loop(0, C, body, 0)
        plsc.subcore_barrier()

        pltpu.sync_copy(acc.at[pl.ds(r0, ROWS_PER_TILE)],
                        part_hbm.at[cid, pl.ds(r0, ROWS_PER_TILE)])

        @pl.when(sid == NUM_SUBCORES - 1)
        def _():
            t0 = NUM_SUBCORES * ROWS_PER_TILE
            pltpu.sync_copy(acc.at[pl.ds(t0, TAIL_ROWS)],
                            part_hbm.at[cid, pl.ds(t0, TAIL_ROWS)])

    return k(x, rows3, cols3, zeros)


def _tc_combine_matmul(partials, W):
    BLK = 1000

    def body(p_ref, w_ref, o_ref):
        s = p_ref[0] + p_ref[1]
        o_ref[...] = lax.dot_general(
            s, w_ref[...], (((1,), (1,)), ((), ())),
            preferred_element_type=jnp.float32)

    return pl.pallas_call(
        body,
        grid=(N // BLK,),
        in_specs=[
            pl.BlockSpec((NUM_CORES, BLK, D), lambda i: (0, i, 0)),
            pl.BlockSpec((D, D), lambda i: (0, 0)),
        ],
        out_specs=pl.BlockSpec((BLK, D), lambda i: (i, 0)),
        out_shape=jax.ShapeDtypeStruct((N, D), jnp.float32),
    )(partials, W)


@jax.jit
def kernel(x, edge_index, W):
    rows3 = edge_index[0].reshape(NW, C, K)
    cols3 = edge_index[1].reshape(NW, C, K)
    zeros = jnp.zeros((N, D), dtype=jnp.float32)
    partials = _sc_aggregate(x, rows3, cols3, zeros)
    return _tc_combine_matmul(partials, W)


# speedup vs baseline: 11.0138x; 1.4094x over previous
import functools

import jax
import jax.numpy as jnp
from jax import lax
from jax.experimental import pallas as pl
from jax.experimental.pallas import tpu as pltpu
from jax.experimental.pallas import tpu_sc as plsc

N = 10000
E = 320000
D = 128

NUM_CORES = 2
NUM_SUBCORES = 16
NW = NUM_CORES * NUM_SUBCORES
EPW = E // NW
K = 125
C = EPW // K
H = 2
CH = C // H
ROWS_PER_TILE = 624
TAIL_ROWS = N - NUM_SUBCORES * ROWS_PER_TILE


def _sc_aggregate(x, rows3, cols3, zeros):
    mesh = plsc.VectorSubcoreMesh(core_axis_name="c", subcore_axis_name="s")

    @functools.partial(
        pl.kernel,
        out_type=jax.ShapeDtypeStruct((NUM_CORES, N, D), jnp.float32),
        mesh=mesh,
        scratch_types=[
            pltpu.VMEM((CH, K), jnp.int32),
            pltpu.VMEM((CH, K), jnp.int32),
            pltpu.VMEM((K, D), jnp.float32),
            pltpu.VMEM((K, D), jnp.float32),
            pltpu.VMEM_SHARED((N, D), jnp.float32),
            pltpu.SemaphoreType.DMA,
            pltpu.SemaphoreType.DMA,
        ],
    )
    def k(x_hbm, rows_hbm, cols_hbm, zeros_hbm, part_hbm,
          colv, rowv, gbuf0, gbuf1, acc, sem0, sem1):
        cid = lax.axis_index("c")
        sid = lax.axis_index("s")
        wid = cid * NUM_SUBCORES + sid

        r0 = sid * ROWS_PER_TILE
        pltpu.sync_copy(zeros_hbm.at[pl.ds(r0, ROWS_PER_TILE)],
                        acc.at[pl.ds(r0, ROWS_PER_TILE)])

        @pl.when(sid == NUM_SUBCORES - 1)
        def _():
            t0 = NUM_SUBCORES * ROWS_PER_TILE
            pltpu.sync_copy(zeros_hbm.at[pl.ds(t0, TAIL_ROWS)],
                            acc.at[pl.ds(t0, TAIL_ROWS)])

        plsc.subcore_barrier()

        def half(h, carry):
            pltpu.sync_copy(cols_hbm.at[wid, h], colv)
            pltpu.sync_copy(rows_hbm.at[wid, h], rowv)

            pltpu.async_copy(x_hbm.at[colv.at[0]], gbuf0, sem0)

            def body(i, carry):
                j0 = 2 * i
                j1 = j0 + 1
                pltpu.async_copy(x_hbm.at[colv.at[j1]], gbuf1, sem1)
                pltpu.make_async_copy(x_hbm.at[colv.at[j0]], gbuf0,
                                      sem0).wait()
                pltpu.sync_copy(gbuf0, acc.at[rowv.at[j0]], add=True)

                @pl.when(i < CH // 2 - 1)
                def _():
                    pltpu.async_copy(x_hbm.at[colv.at[j1 + 1]], gbuf0, sem0)

                pltpu.make_async_copy(x_hbm.at[colv.at[j1]], gbuf1,
                                      sem1).wait()
                pltpu.sync_copy(gbuf1, acc.at[rowv.at[j1]], add=True)
                return carry

            return lax.fori_loop(0, CH // 2, body, carry)

        lax.fori_loop(0, H, half, 0)
        plsc.subcore_barrier()

        pltpu.sync_copy(acc.at[pl.ds(r0, ROWS_PER_TILE)],
                        part_hbm.at[cid, pl.ds(r0, ROWS_PER_TILE)])

        @pl.when(sid == NUM_SUBCORES - 1)
        def _():
            t0 = NUM_SUBCORES * ROWS_PER_TILE
            pltpu.sync_copy(acc.at[pl.ds(t0, TAIL_ROWS)],
                            part_hbm.at[cid, pl.ds(t0, TAIL_ROWS)])

    return k(x, rows3, cols3, zeros)


def _tc_combine_matmul(partials, W):
    BLK = 1000

    def body(p_ref, w_ref, o_ref):
        s = p_ref[0] + p_ref[1]
        o_ref[...] = lax.dot_general(
            s, w_ref[...], (((1,), (1,)), ((), ())),
            preferred_element_type=jnp.float32)

    return pl.pallas_call(
        body,
        grid=(N // BLK,),
        in_specs=[
            pl.BlockSpec((NUM_CORES, BLK, D), lambda i: (0, i, 0)),
            pl.BlockSpec((D, D), lambda i: (0, 0)),
        ],
        out_specs=pl.BlockSpec((BLK, D), lambda i: (i, 0)),
        out_shape=jax.ShapeDtypeStruct((N, D), jnp.float32),
    )(partials, W)


@jax.jit
def kernel(x, edge_index, W):
    rows3 = edge_index[0].reshape(NW, H, CH, K)
    cols3 = edge_index[1].reshape(NW, H, CH, K)
    zeros = jnp.zeros((N, D), dtype=jnp.float32)
    partials = _sc_aggregate(x, rows3, cols3, zeros)
    return _tc_combine_matmul(partials, W)
